# Initial kernel scaffold; baseline (speedup 1.0000x reference)
import jax
import jax.numpy as jnp
from jax.experimental import pallas as pl


def kernel(inputs, edge_index, dropout_mask, W1, b1, W2, b2):
    raise NotImplementedError("write your pallas kernel here")


# speedup vs baseline: 11.0880x; 11.0880x over previous
import functools

import jax
import jax.numpy as jnp
from jax import lax
from jax.experimental import pallas as pl
from jax.experimental.pallas import tpu as pltpu
from jax.experimental.pallas import tpu_sc as plsc

NC = 2
NS = 16
NW = NC * NS
CH = 128


def _fill(ref, val):
    rows, w = ref.shape

    def body(i, c):
        for j in range(w // 16):
            ref[i, pl.ds(j * 16, 16)] = jnp.full((16,), val, jnp.float32)
        return c

    lax.fori_loop(0, rows, body, 0)


def _make_degree(np_rows, epw_chunks):
    mesh = plsc.VectorSubcoreMesh(
        core_axis_name="c", subcore_axis_name="s", num_cores=NC, num_subcores=NS)
    rpt = np_rows // NS
    ndump = rpt // CH

    @functools.partial(
        pl.kernel,
        out_type=jax.ShapeDtypeStruct((NC, np_rows, 16), jnp.float32),
        mesh=mesh,
        scratch_types=[
            pltpu.VMEM((CH,), jnp.int32),
            pltpu.VMEM((CH, 16), jnp.float32),
            pltpu.VMEM_SHARED((np_rows, 16), jnp.float32),
        ],
        compiler_params=pltpu.CompilerParams(use_tc_tiling_on_sc=False),
    )
    def degree(dst_hbm, out_hbm, dst_v, buf_v, acc_sh):
        cid = lax.axis_index("c")
        sid = lax.axis_index("s")
        w = cid * NS + sid
        _fill(buf_v, 0.0)
        for m in range(ndump):
            pltpu.sync_copy(buf_v, acc_sh.at[pl.ds(sid * rpt + m * CH, CH)])
        plsc.subcore_barrier()
        _fill(buf_v, 1.0)
        base0 = w * (epw_chunks * CH)

        def body(j, c):
            base = base0 + j * CH
            pltpu.sync_copy(dst_hbm.at[pl.ds(base, CH)], dst_v)
            pltpu.sync_copy(buf_v, acc_sh.at[dst_v], add=True)
            return c

        lax.fori_loop(0, epw_chunks, body, 0)
        plsc.subcore_barrier()
        for m in range(ndump):
            r0 = sid * rpt + m * CH
            pltpu.sync_copy(acc_sh.at[pl.ds(r0, CH)], buf_v)
            pltpu.sync_copy(buf_v, out_hbm.at[cid, pl.ds(r0, CH)])

    return degree


def _make_agg(np_rows, width, epw_chunks):
    mesh = plsc.VectorSubcoreMesh(
        core_axis_name="c", subcore_axis_name="s", num_cores=NC, num_subcores=NS)
    rpt = np_rows // NS
    ndump = rpt // CH

    @functools.partial(
        pl.kernel,
        out_type=jax.ShapeDtypeStruct((NC, np_rows, width), jnp.float32),
        mesh=mesh,
        scratch_types=[
            pltpu.VMEM((CH,), jnp.int32),
            pltpu.VMEM((CH,), jnp.int32),
            pltpu.VMEM((CH, width), jnp.float32),
            pltpu.VMEM_SHARED((np_rows, width), jnp.float32),
            pltpu.SemaphoreType.DMA,
        ],
        compiler_params=pltpu.CompilerParams(use_tc_tiling_on_sc=False),
    )
    def agg(src_hbm, dst_hbm, y_hbm, out_hbm, src_v, dst_v, rows_v, z_sh, sem):
        cid = lax.axis_index("c")
        sid = lax.axis_index("s")
        w = cid * NS + sid
        _fill(rows_v, 0.0)
        for m in range(ndump):
            pltpu.sync_copy(rows_v, z_sh.at[pl.ds(sid * rpt + m * CH, CH)])
        plsc.subcore_barrier()
        base0 = w * (epw_chunks * CH)

        def body(j, c):
            base = base0 + j * CH
            pltpu.sync_copy(src_hbm.at[pl.ds(base, CH)], src_v)
            pltpu.sync_copy(dst_hbm.at[pl.ds(base, CH)], dst_v)
            pltpu.async_copy(y_hbm.at[src_v], rows_v, sem).wait()
            pltpu.sync_copy(rows_v, z_sh.at[dst_v], add=True)
            return c

        lax.fori_loop(0, epw_chunks, body, 0)
        plsc.subcore_barrier()
        for m in range(ndump):
            r0 = sid * rpt + m * CH
            pltpu.sync_copy(z_sh.at[pl.ds(r0, CH)], rows_v)
            pltpu.sync_copy(rows_v, out_hbm.at[cid, pl.ds(r0, CH)])

    return agg


def _tc_scale_matmul(degp, x_p, w1, np_rows, blk=1024):
    def body(degp_ref, x_ref, w_ref, dinv_ref, y_ref):
        deg = degp_ref[0] + degp_ref[1] + 1.0
        dv = lax.rsqrt(deg)
        dinv_ref[...] = dv
        xw = jnp.dot(x_ref[...], w_ref[...], preferred_element_type=jnp.float32)
        y_ref[...] = xw * dv[:, 0:1]

    g = np_rows // blk
    return pl.pallas_call(
        body,
        grid=(g,),
        in_specs=[
            pl.BlockSpec((NC, blk, 16), lambda i: (0, i, 0)),
            pl.BlockSpec((blk, 128), lambda i: (i, 0)),
            pl.BlockSpec((128, 128), lambda i: (0, 0)),
        ],
        out_specs=[
            pl.BlockSpec((blk, 16), lambda i: (i, 0)),
            pl.BlockSpec((blk, 128), lambda i: (i, 0)),
        ],
        out_shape=[
            jax.ShapeDtypeStruct((np_rows, 16), jnp.float32),
            jax.ShapeDtypeStruct((np_rows, 128), jnp.float32),
        ],
    )(degp, x_p, w1)


def _tc_layer1_finish(z1p, y1, dinv, b1, mask_p, w2p, np_rows, blk=1024):
    def body(z_ref, y1_ref, dv_ref, b1_ref, m_ref, w2_ref, y2_ref):
        dvc = dv_ref[...][:, 0:1]
        z = z_ref[0] + z_ref[1] + y1_ref[...]
        o1 = z * dvc + b1_ref[...]
        h = jnp.maximum(o1, 0.0) * m_ref[...].astype(jnp.float32) * 2.0
        hw = jnp.dot(h, w2_ref[...], preferred_element_type=jnp.float32)
        y2_ref[...] = hw * dvc

    g = np_rows // blk
    return pl.pallas_call(
        body,
        grid=(g,),
        in_specs=[
            pl.BlockSpec((NC, blk, 128), lambda i: (0, i, 0)),
            pl.BlockSpec((blk, 128), lambda i: (i, 0)),
            pl.BlockSpec((blk, 16), lambda i: (i, 0)),
            pl.BlockSpec((1, 128), lambda i: (0, 0)),
            pl.BlockSpec((blk, 128), lambda i: (i, 0)),
            pl.BlockSpec((128, 128), lambda i: (0, 0)),
        ],
        out_specs=pl.BlockSpec((blk, 128), lambda i: (i, 0)),
        out_shape=jax.ShapeDtypeStruct((np_rows, 128), jnp.float32),
    )(z1p, y1, dinv, b1, mask_p, w2p)


def _tc_layer2_finish(z2p, y2, dinv, b2p, np_rows, blk=1024):
    def body(z_ref, y2_ref, dv_ref, b2_ref, o_ref):
        dvc = dv_ref[...][:, 0:1]
        z = z_ref[0] + z_ref[1] + y2_ref[...]
        o_ref[...] = z * dvc + b2_ref[...]

    g = np_rows // blk
    return pl.pallas_call(
        body,
        grid=(g,),
        in_specs=[
            pl.BlockSpec((NC, blk, 128), lambda i: (0, i, 0)),
            pl.BlockSpec((blk, 128), lambda i: (i, 0)),
            pl.BlockSpec((blk, 16), lambda i: (i, 0)),
            pl.BlockSpec((1, 128), lambda i: (0, 0)),
        ],
        out_specs=pl.BlockSpec((blk, 128), lambda i: (i, 0)),
        out_shape=jax.ShapeDtypeStruct((np_rows, 128), jnp.float32),
    )(z2p, y2, dinv, b2p)


def kernel(inputs, edge_index, dropout_mask, W1, b1, W2, b2):
    xs = inputs[0]
    ei = edge_index[0]
    mask = dropout_mask[0]
    n, in_dim = xs.shape
    e = ei.shape[1]
    ncls = W2.shape[1]

    np_rows = -(-(n + 1) // (NS * CH)) * (NS * CH)
    epw_chunks = -(-e // (NW * CH))
    e_pad = NW * CH * epw_chunks

    src = jnp.concatenate([ei[0], jnp.zeros((e_pad - e,), jnp.int32)])
    dst = jnp.concatenate([ei[1], jnp.full((e_pad - e,), n, jnp.int32)])
    x_p = jnp.pad(xs, ((0, np_rows - n), (0, 0)))
    mask_p = jnp.pad(mask, ((0, np_rows - n), (0, 0)))
    w2p = jnp.pad(W2, ((0, 0), (0, 128 - ncls)))
    b1r = b1.reshape(1, -1)
    b2p = jnp.pad(b2, (0, 128 - ncls)).reshape(1, -1)

    degp = _make_degree(np_rows, epw_chunks)(dst)
    dinv, y1 = _tc_scale_matmul(degp, x_p, W1, np_rows)
    z1p = _make_agg(np_rows, 128, epw_chunks)(src, dst, y1)
    y2 = _tc_layer1_finish(z1p, y1, dinv, b1r, mask_p, w2p, np_rows)
    z2p = _make_agg(np_rows, 128, epw_chunks)(src, dst, y2)
    out64 = _tc_layer2_finish(z2p, y2, dinv, b2p, np_rows)
    return out64[:n, :ncls][None]


# speedup vs baseline: 12.2048x; 1.1007x over previous
import functools

import jax
import jax.numpy as jnp
from jax import lax
from jax.experimental import pallas as pl
from jax.experimental.pallas import tpu as pltpu
from jax.experimental.pallas import tpu_sc as plsc

NC = 2
NS = 16
NW = NC * NS
CH = 128
HW = 64


def _fill(ref, val):
    rows, w = ref.shape

    def body(i, c):
        for j in range(w // 16):
            ref[i, pl.ds(j * 16, 16)] = jnp.full((16,), val, jnp.float32)
        return c

    lax.fori_loop(0, rows, body, 0)


def _make_degree(np_rows, cpw):
    mesh = plsc.VectorSubcoreMesh(
        core_axis_name="c", subcore_axis_name="s", num_cores=NC, num_subcores=NS)
    rpt = np_rows // NS
    ndump = rpt // CH

    @functools.partial(
        pl.kernel,
        out_type=jax.ShapeDtypeStruct((NC, np_rows, 16), jnp.float32),
        mesh=mesh,
        scratch_types=[
            pltpu.VMEM((CH,), jnp.int32),
            pltpu.VMEM((CH, 16), jnp.float32),
            pltpu.VMEM_SHARED((np_rows, 16), jnp.float32),
        ],
        compiler_params=pltpu.CompilerParams(use_tc_tiling_on_sc=False),
    )
    def degree(dst_hbm, out_hbm, dst_v, buf_v, acc_sh):
        cid = lax.axis_index("c")
        sid = lax.axis_index("s")
        w = cid * NS + sid
        _fill(buf_v, 0.0)
        for m in range(ndump):
            pltpu.sync_copy(buf_v, acc_sh.at[pl.ds(sid * rpt + m * CH, CH)])
        plsc.subcore_barrier()
        _fill(buf_v, 1.0)
        base0 = w * cpw

        def body(j, c):
            pltpu.sync_copy(dst_hbm.at[base0 + j], dst_v)
            pltpu.sync_copy(buf_v, acc_sh.at[dst_v], add=True)
            return c

        lax.fori_loop(0, cpw, body, 0)
        plsc.subcore_barrier()
        for m in range(ndump):
            r0 = sid * rpt + m * CH
            pltpu.sync_copy(acc_sh.at[pl.ds(r0, CH)], buf_v)
            pltpu.sync_copy(buf_v, out_hbm.at[cid, pl.ds(r0, CH)])

    return degree


def _make_agg(np_rows, cpt):
    mesh = plsc.VectorSubcoreMesh(
        core_axis_name="c", subcore_axis_name="s", num_cores=NC, num_subcores=NS)
    rpt = np_rows // NS
    ndump = rpt // CH
    assert cpt % 4 == 0 and cpt >= 12

    @functools.partial(
        pl.kernel,
        out_type=jax.ShapeDtypeStruct((NC, np_rows, HW), jnp.float32),
        mesh=mesh,
        scratch_types=[
            pltpu.VMEM((cpt, CH), jnp.int32),
            pltpu.VMEM((cpt, CH), jnp.int32),
            pltpu.VMEM((CH, HW), jnp.float32),
            pltpu.VMEM((CH, HW), jnp.float32),
            pltpu.VMEM((CH, HW), jnp.float32),
            pltpu.VMEM((CH, HW), jnp.float32),
            pltpu.VMEM((CH,), jnp.int32),
            pltpu.VMEM((CH,), jnp.int32),
            pltpu.VMEM((CH,), jnp.int32),
            pltpu.VMEM((CH,), jnp.int32),
            pltpu.VMEM((CH,), jnp.int32),
            pltpu.VMEM((CH,), jnp.int32),
            pltpu.VMEM((CH,), jnp.int32),
            pltpu.VMEM((CH,), jnp.int32),
            pltpu.VMEM_SHARED((np_rows, HW), jnp.float32),
            pltpu.SemaphoreType.DMA,
            pltpu.SemaphoreType.DMA,
            pltpu.SemaphoreType.DMA,
            pltpu.SemaphoreType.DMA,
            pltpu.SemaphoreType.DMA,
            pltpu.SemaphoreType.DMA,
            pltpu.SemaphoreType.DMA,
            pltpu.SemaphoreType.DMA,
            pltpu.SemaphoreType.DMA,
        ],
        compiler_params=pltpu.CompilerParams(use_tc_tiling_on_sc=False),
    )
    def agg(src_hbm, dst_hbm, y_hbm, out_hbm, src_v, dst_v,
            b0, b1, b2, b3, a0, a1, a2, a3, d0, d1, d2, d3, z_sh,
            g0, g1, g2, g3, s0, s1, s2, s3, isem):
        bufs = (b0, b1, b2, b3)
        abufs = (a0, a1, a2, a3)
        dbufs = (d0, d1, d2, d3)
        gsems = (g0, g1, g2, g3)
        ssems = (s0, s1, s2, s3)
        cid = lax.axis_index("c")
        sid = lax.axis_index("s")
        coff = jnp.zeros((16,), jnp.int32) + cid * np_rows
        r0w = sid * cpt
        pltpu.async_copy(src_hbm.at[pl.ds(r0w, cpt)], src_v, isem)
        pltpu.async_copy(dst_hbm.at[pl.ds(r0w, cpt)], dst_v, isem)
        _fill(b0, 0.0)
        for m in range(ndump):
            pltpu.sync_copy(b0, z_sh.at[pl.ds(sid * rpt + m * CH, CH)])
        plsc.subcore_barrier()
        pltpu.make_async_copy(
            src_hbm.at[pl.ds(r0w, cpt)], src_v, isem).wait()
        pltpu.make_async_copy(
            dst_hbm.at[pl.ds(r0w, cpt)], dst_v, isem).wait()

        def gather(j, b):
            for k in range(CH // 16):
                abufs[b][pl.ds(k * 16, 16)] = (
                    src_v[j, pl.ds(k * 16, 16)] + coff)
            pltpu.async_copy(y_hbm.at[abufs[b]], bufs[b], gsems[b])

        def gather_wait(b):
            pltpu.make_async_copy(
                y_hbm.at[abufs[b]], bufs[b], gsems[b]).wait()

        def scatter(j, b):
            for k in range(CH // 16):
                dbufs[b][pl.ds(k * 16, 16)] = dst_v[j, pl.ds(k * 16, 16)]
            pltpu.async_copy(bufs[b], z_sh.at[dbufs[b]], ssems[b], add=True)

        def scatter_wait(b):
            pltpu.make_async_copy(
                bufs[b], z_sh.at[dbufs[b]], ssems[b]).wait()

        gather(0, 0)
        gather(1, 1)
        for b in range(4):
            if b >= 2:
                scatter_wait((b + 2) % 4)
            gather(b + 2, (b + 2) % 4)
            gather_wait(b)
            scatter(b, b)

        def group(g, c):
            j0 = g * 4
            for b in range(4):
                scatter_wait((b + 2) % 4)
                gather(j0 + b + 2, (b + 2) % 4)
                gather_wait(b)
                scatter(j0 + b, b)
            return c

        lax.fori_loop(1, cpt // 4 - 1, group, 0)
        j0 = cpt - 4
        for b in range(4):
            jj = j0 + b + 2
            if jj < cpt:
                scatter_wait((b + 2) % 4)
                gather(jj, (b + 2) % 4)
            gather_wait(b)
            scatter(j0 + b, b)
        for b in range(4):
            scatter_wait(b)
        plsc.subcore_barrier()
        for m in range(ndump):
            r0 = sid * rpt + m * CH
            pltpu.sync_copy(z_sh.at[pl.ds(r0, CH)], b0)
            pltpu.sync_copy(b0, out_hbm.at[cid, pl.ds(r0, CH)])

    return agg


def _tc_scale_matmul(degp, x_p, w1, np_rows, blk=1024):
    def body(degp_ref, x_ref, w_ref, dinv_ref, y_ref):
        deg = degp_ref[0] + degp_ref[1] + 1.0
        dv = lax.rsqrt(deg)
        dinv_ref[...] = dv
        xw = jnp.dot(x_ref[...], w_ref[...], preferred_element_type=jnp.float32)
        xw = xw * dv[:, 0:1]
        y_ref[0] = xw[:, :HW]
        y_ref[1] = xw[:, HW:]

    g = np_rows // blk
    return pl.pallas_call(
        body,
        grid=(g,),
        in_specs=[
            pl.BlockSpec((NC, blk, 16), lambda i: (0, i, 0)),
            pl.BlockSpec((blk, 128), lambda i: (i, 0)),
            pl.BlockSpec((128, 128), lambda i: (0, 0)),
        ],
        out_specs=[
            pl.BlockSpec((blk, 16), lambda i: (i, 0)),
            pl.BlockSpec((NC, blk, HW), lambda i: (0, i, 0)),
        ],
        out_shape=[
            jax.ShapeDtypeStruct((np_rows, 16), jnp.float32),
            jax.ShapeDtypeStruct((NC, np_rows, HW), jnp.float32),
        ],
    )(degp, x_p, w1)


def _tc_layer1_finish(z1p, y1sp, dinv, b1, mask_p, w2p, np_rows, blk=1024):
    def body(z_ref, y1_ref, dv_ref, b1_ref, m_ref, w2_ref, y2_ref):
        dvc = dv_ref[...][:, 0:1]
        z = jnp.concatenate([z_ref[0], z_ref[1]], axis=1)
        y1 = jnp.concatenate([y1_ref[0], y1_ref[1]], axis=1)
        o1 = (z + y1) * dvc + b1_ref[...]
        h = jnp.maximum(o1, 0.0) * m_ref[...].astype(jnp.float32) * 2.0
        hw = jnp.dot(h, w2_ref[...], preferred_element_type=jnp.float32)
        hw = hw * dvc
        y2_ref[0] = hw[:, :HW]
        y2_ref[1] = hw[:, HW:]

    g = np_rows // blk
    return pl.pallas_call(
        body,
        grid=(g,),
        in_specs=[
            pl.BlockSpec((NC, blk, HW), lambda i: (0, i, 0)),
            pl.BlockSpec((NC, blk, HW), lambda i: (0, i, 0)),
            pl.BlockSpec((blk, 16), lambda i: (i, 0)),
            pl.BlockSpec((1, 128), lambda i: (0, 0)),
            pl.BlockSpec((blk, 128), lambda i: (i, 0)),
            pl.BlockSpec((128, 128), lambda i: (0, 0)),
        ],
        out_specs=pl.BlockSpec((NC, blk, HW), lambda i: (0, i, 0)),
        out_shape=jax.ShapeDtypeStruct((NC, np_rows, HW), jnp.float32),
    )(z1p, y1sp, dinv, b1, mask_p, w2p)


def _tc_layer2_finish(z2p, y2sp, dinv, b2p, np_rows, blk=1024):
    def body(z_ref, y2_ref, dv_ref, b2_ref, o_ref):
        dvc = dv_ref[...][:, 0:1]
        z = jnp.concatenate([z_ref[0], z_ref[1]], axis=1)
        y2 = jnp.concatenate([y2_ref[0], y2_ref[1]], axis=1)
        o_ref[...] = (z + y2) * dvc + b2_ref[...]

    g = np_rows // blk
    return pl.pallas_call(
        body,
        grid=(g,),
        in_specs=[
            pl.BlockSpec((NC, blk, HW), lambda i: (0, i, 0)),
            pl.BlockSpec((NC, blk, HW), lambda i: (0, i, 0)),
            pl.BlockSpec((blk, 16), lambda i: (i, 0)),
            pl.BlockSpec((1, 128), lambda i: (0, 0)),
        ],
        out_specs=pl.BlockSpec((blk, 128), lambda i: (i, 0)),
        out_shape=jax.ShapeDtypeStruct((np_rows, 128), jnp.float32),
    )(z2p, y2sp, dinv, b2p)


def kernel(inputs, edge_index, dropout_mask, W1, b1, W2, b2):
    xs = inputs[0]
    ei = edge_index[0]
    mask = dropout_mask[0]
    n, in_dim = xs.shape
    e = ei.shape[1]
    ncls = W2.shape[1]

    np_rows = -(-(n + 1) // (NS * CH)) * (NS * CH)
    cpt = -(-e // (NS * CH))
    cpt = -(-cpt // 4) * 4
    e_pad = NS * CH * cpt
    cpw_deg = e_pad // (NW * CH)

    src = jnp.concatenate([ei[0], jnp.zeros((e_pad - e,), jnp.int32)])
    dst = jnp.concatenate([ei[1], jnp.full((e_pad - e,), n, jnp.int32)])
    src = src.reshape(e_pad // CH, CH)
    dst = dst.reshape(e_pad // CH, CH)
    x_p = jnp.pad(xs, ((0, np_rows - n), (0, 0)))
    mask_p = jnp.pad(mask, ((0, np_rows - n), (0, 0)))
    w2p = jnp.pad(W2, ((0, 0), (0, 128 - ncls)))
    b1r = b1.reshape(1, -1)
    b2p = jnp.pad(b2, (0, 128 - ncls)).reshape(1, -1)

    agg = _make_agg(np_rows, cpt)
    degp = _make_degree(np_rows, cpw_deg)(dst)
    dinv, y1sp = _tc_scale_matmul(degp, x_p, W1, np_rows)
    z1p = agg(src, dst, y1sp.reshape(NC * np_rows, HW))
    y2sp = _tc_layer1_finish(z1p, y1sp, dinv, b1r, mask_p, w2p, np_rows)
    z2p = agg(src, dst, y2sp.reshape(NC * np_rows, HW))
    out = _tc_layer2_finish(z2p, y2sp, dinv, b2p, np_rows)
    return out[:n, :ncls][None]
